# baseline (speedup 1.0000x reference)
import functools
from math import ceil

import jax
import jax.numpy as jnp
from jax import lax
from jax.experimental import pallas as pl
from jax.experimental.pallas import tpu as pltpu
from jax.experimental.pallas import tpu_sc as plsc

_B, _N, _P = 4, 512, 16384
_LANES = 128
_NUM = ceil(0.9 * (_N - 1))
_K = (_N - 1) - _NUM
_L = 16
_NV = _N // _L

_NT = 384
_RB = 128
_NSC = _N - _NT
_TPB = 8
_RPT = _NSC // _TPB
_CR = 2
_NCH = _RPT // _CR
_UNROLL = 4
_NTV = _NT // _L


def _mv_body(dens_ref, prob_ref, out_ref):
    b = pl.program_id(0)
    r = pl.program_id(1)
    blk = prob_ref[0]
    d = dens_ref[pl.ds(b, 1)]
    acc = jnp.zeros((_RB, _LANES), jnp.float32)
    for j in range(_P // _LANES):
        sl = slice(j * _LANES, (j + 1) * _LANES)
        acc = acc + blk[:, sl] * d[:, sl]
    out_ref[0, 0, pl.ds(r * _RB, _RB)] = jnp.sum(acc, axis=1)


def _matvec(prob_list, pre_density):
    return pl.pallas_call(
        _mv_body,
        grid=(_B, _NT // _RB),
        in_specs=[
            pl.BlockSpec((_B, _P), lambda b, r: (0, 0)),
            pl.BlockSpec((1, _RB, _P), lambda b, r: (b, r, 0)),
        ],
        out_specs=pl.BlockSpec((1, 1, _NT), lambda b, r: (b, 0, 0)),
        out_shape=jax.ShapeDtypeStruct((_B, 1, _NT), jnp.float32),
    )(pre_density, prob_list)


def _gather16(v, idx):
    return lax.gather(
        v,
        idx[:, None],
        lax.GatherDimensionNumbers(
            offset_dims=(), collapsed_slice_dims=(0,), start_index_map=(0,)
        ),
        (1,),
        mode=lax.GatherScatterMode.PROMISE_IN_BOUNDS,
    )


def _xlane_sum(v):
    lane = lax.iota(jnp.int32, _L)
    for s in (1, 2, 4, 8):
        v = v + _gather16(v, lane ^ s)
    return v


def _sc_mv_body(
    prob_hbm, dens_hbm, out_hbm, dens_v, buf0, buf1, buf2, sums_v, sem0, sem1, sem2
):
    cid = lax.axis_index("c")
    sid = lax.axis_index("s")
    wid = sid * 2 + cid
    b = wid // _TPB
    r0 = _NT + (wid % _TPB) * _RPT
    lane = lax.iota(jnp.int32, _L)
    lane0 = lane == 0

    pltpu.sync_copy(dens_hbm.at[b], dens_v)
    bufs = (buf0, buf1, buf2)
    sems = (sem0, sem1, sem2)
    nbuf = len(bufs)
    copies = [None] * nbuf

    def _issue(c):
        return pltpu.async_copy(
            prob_hbm.at[b, pl.ds(r0 + c * _CR, _CR)],
            bufs[c % nbuf],
            sems[c % nbuf],
        )

    for c in range(nbuf - 1):
        copies[c] = _issue(c)
    for c in range(_NCH):
        cur = c % nbuf
        if c + nbuf - 1 < _NCH:
            copies[(c + nbuf - 1) % nbuf] = _issue(c + nbuf - 1)
        copies[cur].wait()
        buf = bufs[cur]

        def _acc_body(v, accs):
            accs = list(accs)
            for u in range(_UNROLL):
                sl = pl.ds((v * _UNROLL + u) * _L, _L)
                d = dens_v[sl]
                for r in range(_CR):
                    accs[r] = accs[r] + buf[r, sl] * d
            return tuple(accs)

        accs = lax.fori_loop(
            0,
            _P // (_L * _UNROLL),
            _acc_body,
            tuple(jnp.zeros((_L,), jnp.float32) for _ in range(_CR)),
        )
        for r in range(_CR):
            s = _xlane_sum(accs[r])
            plsc.store_scatter(
                sums_v, [jnp.full((_L,), c * _CR + r, jnp.int32)], s, mask=lane0
            )
    pltpu.sync_copy(sums_v, out_hbm.at[b, pl.ds((wid % _TPB) * _RPT, _RPT)])


@functools.cache
def _sc_matvec():
    return pl.kernel(
        _sc_mv_body,
        out_type=jax.ShapeDtypeStruct((_B, _NSC), jnp.float32),
        mesh=plsc.VectorSubcoreMesh(
            core_axis_name="c", subcore_axis_name="s", num_cores=2, num_subcores=16
        ),
        compiler_params=pltpu.CompilerParams(needs_layout_passes=False),
        scratch_types=[
            pltpu.VMEM((_P,), jnp.float32),
            pltpu.VMEM((_CR, _P), jnp.float32),
            pltpu.VMEM((_CR, _P), jnp.float32),
            pltpu.VMEM((_CR, _P), jnp.float32),
            pltpu.VMEM((_RPT,), jnp.float32),
            pltpu.SemaphoreType.DMA,
            pltpu.SemaphoreType.DMA,
            pltpu.SemaphoreType.DMA,
        ],
    )


def _tc_loss_body(pc_ref, ps_ref, tg_ref, out_ref):
    pc = jnp.concatenate([pc_ref[:, 0, :], ps_ref[...]], axis=1)
    tg = jnp.concatenate(
        [tg_ref[...], jnp.zeros((_B, 1), jnp.float32)], axis=1
    )
    res = jnp.abs(tg - pc)
    col = lax.broadcasted_iota(jnp.int32, (_B, _N), 1)
    is_last = col == (_N - 1)
    resm = jnp.where(is_last, jnp.float32(-1.0), res)
    res511 = jnp.sum(jnp.where(is_last, res, 0.0), axis=1, keepdims=True)
    total = jnp.sum(resm, axis=1, keepdims=True) + jnp.float32(1.0)

    def _bs(_, carry):
        lo, hi = carry
        mid = lo + lax.shift_right_logical(hi - lo, 1)
        t = lax.bitcast_convert_type(mid, jnp.float32)
        cnt = jnp.sum((resm > t).astype(jnp.int32), axis=1, keepdims=True)
        pred = cnt > (_K - 1)
        return (jnp.where(pred, mid + 1, lo), jnp.where(pred, hi, mid))

    lo, _ = lax.fori_loop(
        0,
        31,
        _bs,
        (
            jnp.zeros((_B, 1), jnp.int32),
            jnp.full((_B, 1), 0x7F800000, jnp.int32),
        ),
    )
    tv = lax.bitcast_convert_type(lo, jnp.float32)
    m = resm > tv
    sum_gt = jnp.sum(jnp.where(m, resm, 0.0), axis=1, keepdims=True)
    cnt_gt = jnp.sum(m.astype(jnp.int32), axis=1, keepdims=True)
    sum_top = sum_gt + (_K - cnt_gt).astype(jnp.float32) * tv
    loss = total - sum_top + res511
    out_ref[...] = jnp.sum(loss, keepdims=True).reshape(1, 1) / _B


def _tc_loss(pc_tc, pc_sc, target_list):
    return pl.pallas_call(
        _tc_loss_body,
        in_specs=[
            pl.BlockSpec((_B, 1, _NT), lambda: (0, 0, 0)),
            pl.BlockSpec((_B, _NSC), lambda: (0, 0)),
            pl.BlockSpec((_B, _N - 1), lambda: (0, 0)),
        ],
        out_specs=pl.BlockSpec((1, 1), lambda: (0, 0)),
        out_shape=jax.ShapeDtypeStruct((1, 1), jnp.float32),
    )(pc_tc, pc_sc, target_list)


def kernel(prob_list, target_list, pre_density):
    pc_sc = _sc_matvec()(prob_list, pre_density)
    pc_tc = _matvec(prob_list, pre_density)
    return _tc_loss(pc_tc, pc_sc, target_list)[0, 0]


# speedup vs baseline: 1.0527x; 1.0527x over previous
import functools
from math import ceil

import jax
import jax.numpy as jnp
from jax import lax
from jax.experimental import pallas as pl
from jax.experimental.pallas import tpu as pltpu
from jax.experimental.pallas import tpu_sc as plsc

_B, _N, _P = 4, 512, 16384
_LANES = 128
_NUM = ceil(0.9 * (_N - 1))
_K = (_N - 1) - _NUM
_L = 16
_NV = _N // _L

_NT = 512
_RB = 128
_NSC = _N - _NT
_TPB = 8
_RPT = _NSC // _TPB
_CR = 2
_NCH = _RPT // _CR
_UNROLL = 4
_NTV = _NT // _L


def _mv_body(dens_ref, prob_ref, out_ref):
    b = pl.program_id(0)
    r = pl.program_id(1)
    blk = prob_ref[0]
    d = dens_ref[pl.ds(b, 1)]
    acc = jnp.zeros((_RB, _LANES), jnp.float32)
    for j in range(_P // _LANES):
        sl = slice(j * _LANES, (j + 1) * _LANES)
        acc = acc + blk[:, sl] * d[:, sl]
    out_ref[0, 0, pl.ds(r * _RB, _RB)] = jnp.sum(acc, axis=1)


def _matvec(prob_list, pre_density):
    return pl.pallas_call(
        _mv_body,
        grid=(_B, _NT // _RB),
        in_specs=[
            pl.BlockSpec((_B, _P), lambda b, r: (0, 0)),
            pl.BlockSpec((1, _RB, _P), lambda b, r: (b, r, 0)),
        ],
        out_specs=pl.BlockSpec((1, 1, _NT), lambda b, r: (b, 0, 0)),
        out_shape=jax.ShapeDtypeStruct((_B, 1, _NT), jnp.float32),
    )(pre_density, prob_list)


def _gather16(v, idx):
    return lax.gather(
        v,
        idx[:, None],
        lax.GatherDimensionNumbers(
            offset_dims=(), collapsed_slice_dims=(0,), start_index_map=(0,)
        ),
        (1,),
        mode=lax.GatherScatterMode.PROMISE_IN_BOUNDS,
    )


def _xlane_sum(v):
    lane = lax.iota(jnp.int32, _L)
    for s in (1, 2, 4, 8):
        v = v + _gather16(v, lane ^ s)
    return v


def _sc_mv_body(
    prob_hbm, dens_hbm, out_hbm, dens_v, buf0, buf1, buf2, sums_v, sem0, sem1, sem2
):
    cid = lax.axis_index("c")
    sid = lax.axis_index("s")
    wid = sid * 2 + cid
    b = wid // _TPB
    r0 = _NT + (wid % _TPB) * _RPT
    lane = lax.iota(jnp.int32, _L)
    lane0 = lane == 0

    pltpu.sync_copy(dens_hbm.at[b], dens_v)
    bufs = (buf0, buf1, buf2)
    sems = (sem0, sem1, sem2)
    nbuf = len(bufs)
    copies = [None] * nbuf

    def _issue(c):
        return pltpu.async_copy(
            prob_hbm.at[b, pl.ds(r0 + c * _CR, _CR)],
            bufs[c % nbuf],
            sems[c % nbuf],
        )

    for c in range(nbuf - 1):
        copies[c] = _issue(c)
    for c in range(_NCH):
        cur = c % nbuf
        if c + nbuf - 1 < _NCH:
            copies[(c + nbuf - 1) % nbuf] = _issue(c + nbuf - 1)
        copies[cur].wait()
        buf = bufs[cur]

        def _acc_body(v, accs):
            accs = list(accs)
            for u in range(_UNROLL):
                sl = pl.ds((v * _UNROLL + u) * _L, _L)
                d = dens_v[sl]
                for r in range(_CR):
                    accs[r] = accs[r] + buf[r, sl] * d
            return tuple(accs)

        accs = lax.fori_loop(
            0,
            _P // (_L * _UNROLL),
            _acc_body,
            tuple(jnp.zeros((_L,), jnp.float32) for _ in range(_CR)),
        )
        for r in range(_CR):
            s = _xlane_sum(accs[r])
            plsc.store_scatter(
                sums_v, [jnp.full((_L,), c * _CR + r, jnp.int32)], s, mask=lane0
            )
    pltpu.sync_copy(sums_v, out_hbm.at[b, pl.ds((wid % _TPB) * _RPT, _RPT)])


@functools.cache
def _sc_matvec():
    return pl.kernel(
        _sc_mv_body,
        out_type=jax.ShapeDtypeStruct((_B, _NSC), jnp.float32),
        mesh=plsc.VectorSubcoreMesh(
            core_axis_name="c", subcore_axis_name="s", num_cores=2, num_subcores=16
        ),
        compiler_params=pltpu.CompilerParams(needs_layout_passes=False),
        scratch_types=[
            pltpu.VMEM((_P,), jnp.float32),
            pltpu.VMEM((_CR, _P), jnp.float32),
            pltpu.VMEM((_CR, _P), jnp.float32),
            pltpu.VMEM((_CR, _P), jnp.float32),
            pltpu.VMEM((_RPT,), jnp.float32),
            pltpu.SemaphoreType.DMA,
            pltpu.SemaphoreType.DMA,
            pltpu.SemaphoreType.DMA,
        ],
    )


def _sc_loss_body(pc_hbm, ps_hbm, tp_hbm, out_hbm, pc_v, ps_v, tp_v, res_v, out_v):
    wid = lax.axis_index("s")

    @pl.when(wid < _B)
    def _work():
        b = wid
        pltpu.sync_copy(pc_hbm.at[b, 0], pc_v)
        if _NSC:
            pltpu.sync_copy(ps_hbm.at[b], ps_v)
        pltpu.sync_copy(tp_hbm.at[b], tp_v)

        lane = lax.iota(jnp.int32, _L)
        last = lane == (_L - 1)

        def _res_body_tc(j, total_vec):
            sl = pl.ds(j * _L, _L)
            r = jnp.abs(tp_v[sl] - pc_v[sl])
            res_v[sl] = r
            return total_vec + r

        total_vec = lax.fori_loop(
            0, min(_NTV, _NV - 1), _res_body_tc, jnp.zeros((_L,), jnp.float32)
        )

        def _res_body_sc(j, total_vec):
            sl = pl.ds((_NTV + j) * _L, _L)
            r = jnp.abs(tp_v[sl] - ps_v[pl.ds(j * _L, _L)])
            res_v[sl] = r
            return total_vec + r

        if _NSC:
            total_vec = lax.fori_loop(0, _NV - _NTV - 1, _res_body_sc, total_vec)
        sl = pl.ds((_NV - 1) * _L, _L)
        if _NSC:
            lastsrc = ps_v[pl.ds((_NV - _NTV - 1) * _L, _L)]
        else:
            lastsrc = pc_v[sl]
        r = jnp.abs(tp_v[sl] - lastsrc)
        res511_vec = jnp.where(last, r, 0.0)
        r = jnp.where(last, jnp.float32(-1.0), r)
        total_vec = total_vec + jnp.where(last, 0.0, r)
        res_v[sl] = r
        total = _xlane_sum(total_vec)
        res511 = _xlane_sum(res511_vec)

        km1 = jnp.full((_L,), _K - 1, jnp.int32)

        def _bs_body(_, carry):
            lo, hi = carry
            mid = lo + lax.shift_right_logical(hi - lo, 1)
            t = plsc.bitcast(mid, jnp.float32)
            cnt_vec = jnp.zeros((_L,), jnp.int32)
            for j in range(_NV):
                rr = res_v[pl.ds(j * _L, _L)]
                cnt_vec = cnt_vec + jnp.where(rr > t, 1, 0).astype(jnp.int32)
            cnt = _xlane_sum(cnt_vec)
            pred = cnt > km1
            return (jnp.where(pred, mid + 1, lo), jnp.where(pred, hi, mid))

        lo, _ = lax.fori_loop(
            0,
            31,
            _bs_body,
            (
                jnp.zeros((_L,), jnp.int32),
                jnp.full((_L,), 0x7F800000, jnp.int32),
            ),
        )

        tvec = plsc.bitcast(lo, jnp.float32)

        def _gt_body(j, carry):
            sum_gt_vec, cnt_gt_vec = carry
            rr = res_v[pl.ds(j * _L, _L)]
            m = rr > tvec
            return (
                sum_gt_vec + jnp.where(m, rr, 0.0),
                cnt_gt_vec + jnp.where(m, 1, 0).astype(jnp.int32),
            )

        sum_gt_vec, cnt_gt_vec = lax.fori_loop(
            0,
            _NV,
            _gt_body,
            (jnp.zeros((_L,), jnp.float32), jnp.zeros((_L,), jnp.int32)),
        )
        sum_gt = _xlane_sum(sum_gt_vec)
        cnt_gt = _xlane_sum(cnt_gt_vec)
        sum_top = sum_gt + (jnp.full((_L,), _K, jnp.int32) - cnt_gt).astype(
            jnp.float32
        ) * tvec

        out_v[...] = total - sum_top + res511
        pltpu.sync_copy(out_v, out_hbm.at[b])


@functools.cache
def _sc_loss():
    return pl.kernel(
        _sc_loss_body,
        out_type=jax.ShapeDtypeStruct((_B, _L), jnp.float32),
        mesh=plsc.VectorSubcoreMesh(
            core_axis_name="c", subcore_axis_name="s", num_cores=1, num_subcores=16
        ),
        compiler_params=pltpu.CompilerParams(needs_layout_passes=False),
        scratch_types=[
            pltpu.VMEM((_NT,), jnp.float32),
            pltpu.VMEM((max(_NSC, _L),), jnp.float32),
            pltpu.VMEM((_N,), jnp.float32),
            pltpu.VMEM((_N,), jnp.float32),
            pltpu.VMEM((_L,), jnp.float32),
        ],
    )


def kernel(prob_list, target_list, pre_density):
    tpad = jnp.zeros((_B, _N), jnp.float32).at[:, : _N - 1].set(target_list)
    pc_tc = _matvec(prob_list, pre_density)
    if _NSC:
        pc_sc = _sc_matvec()(prob_list, pre_density)
    else:
        pc_sc = pc_tc
    per_batch = _sc_loss()(pc_tc, pc_sc, tpad)
    return jnp.sum(per_batch[:, 0]) / _B
